# Initial kernel scaffold; baseline (speedup 1.0000x reference)
import jax
import jax.numpy as jnp
from jax.experimental import pallas as pl


def kernel(x, edge_index, Wp, bp, Wl0, bl0, Wr0, br0, att0, bias0, Wl1, bl1, Wr1, br1, att1, bias1, Wpred, bpred):
    raise NotImplementedError("write your pallas kernel here")


# speedup vs baseline: 25.8392x; 25.8392x over previous
import functools

import jax
import jax.numpy as jnp
from jax import lax
from jax.experimental import pallas as pl
from jax.experimental.pallas import tpu as pltpu
from jax.experimental.pallas import tpu_sc as plsc

N = 100000
E = 1600000
IN_CH = 128
D = 32
H = 8
C = 4
NUM_CLASS = 7
HALF = 16
ROWW = 20
EPC = 80
NSUB = 16
EPT = E // NSUB
CHUNKS = EPT // EPC
EROWS = E // EPC
NHALF = N // 2
SHROWS = NHALF + 8
ZROWS = 1048
BN = 4000


def _proj_in_body(x_ref, wp_ref, bp_ref, wl_ref, bl_ref, wr_ref, br_ref,
                  h_ref, xl_ref, xr_ref):
    h = jnp.dot(x_ref[...], wp_ref[...],
                preferred_element_type=jnp.float32) + bp_ref[...]
    h_ref[...] = h
    xl = jnp.dot(h, wl_ref[...], preferred_element_type=jnp.float32) + bl_ref[...]
    xr = jnp.dot(h, wr_ref[...], preferred_element_type=jnp.float32) + br_ref[...]
    xl_ref[0] = xl[:, :HALF]
    xl_ref[1] = xl[:, HALF:]
    xr_ref[0] = xr[:, :HALF]
    xr_ref[1] = xr[:, HALF:]


def _proj_in(x, wp, bp, wl, bl, wr, br):
    return pl.pallas_call(
        _proj_in_body,
        grid=(N // BN,),
        in_specs=[
            pl.BlockSpec((BN, IN_CH), lambda i: (i, 0)),
            pl.BlockSpec((IN_CH, D), lambda i: (0, 0)),
            pl.BlockSpec((1, D), lambda i: (0, 0)),
            pl.BlockSpec((D, D), lambda i: (0, 0)),
            pl.BlockSpec((1, D), lambda i: (0, 0)),
            pl.BlockSpec((D, D), lambda i: (0, 0)),
            pl.BlockSpec((1, D), lambda i: (0, 0)),
        ],
        out_specs=[
            pl.BlockSpec((BN, D), lambda i: (i, 0)),
            pl.BlockSpec((2, BN, HALF), lambda i: (0, i, 0)),
            pl.BlockSpec((2, BN, HALF), lambda i: (0, i, 0)),
        ],
        out_shape=[
            jax.ShapeDtypeStruct((N, D), jnp.float32),
            jax.ShapeDtypeStruct((2, N, HALF), jnp.float32),
            jax.ShapeDtypeStruct((2, N, HALF), jnp.float32),
        ],
    )(x, wp, bp, wl, bl, wr, br)


def _rep_matrix():
    gi = lax.broadcasted_iota(jnp.int32, (4, HALF), 0)
    gj = lax.broadcasted_iota(jnp.int32, (4, HALF), 1)
    return (gj // C == gi).astype(jnp.float32)


def _combine(h, acc_ref, bias):
    rep = _rep_matrix()
    a0 = acc_ref[0]
    a1 = acc_ref[1]
    r0 = jnp.dot(1.0 / (a0[:, HALF:ROWW] + 1e-16), rep,
                 preferred_element_type=jnp.float32)
    r1 = jnp.dot(1.0 / (a1[:, HALF:ROWW] + 1e-16), rep,
                 preferred_element_type=jnp.float32)
    out = jnp.concatenate([a0[:, :HALF] * r0, a1[:, :HALF] * r1], axis=1)
    return h + out + bias


def _mid_body(h_ref, acc_ref, bias_ref, wl_ref, bl_ref, wr_ref, br_ref,
              hn_ref, xl_ref, xr_ref):
    hn = _combine(h_ref[...], acc_ref, bias_ref[...])
    hn_ref[...] = hn
    xl = jnp.dot(hn, wl_ref[...], preferred_element_type=jnp.float32) + bl_ref[...]
    xr = jnp.dot(hn, wr_ref[...], preferred_element_type=jnp.float32) + br_ref[...]
    xl_ref[0] = xl[:, :HALF]
    xl_ref[1] = xl[:, HALF:]
    xr_ref[0] = xr[:, :HALF]
    xr_ref[1] = xr[:, HALF:]


def _mid(h, acc, bias, wl, bl, wr, br):
    return pl.pallas_call(
        _mid_body,
        grid=(N // BN,),
        in_specs=[
            pl.BlockSpec((BN, D), lambda i: (i, 0)),
            pl.BlockSpec((2, BN, ROWW), lambda i: (0, i, 0)),
            pl.BlockSpec((1, D), lambda i: (0, 0)),
            pl.BlockSpec((D, D), lambda i: (0, 0)),
            pl.BlockSpec((1, D), lambda i: (0, 0)),
            pl.BlockSpec((D, D), lambda i: (0, 0)),
            pl.BlockSpec((1, D), lambda i: (0, 0)),
        ],
        out_specs=[
            pl.BlockSpec((BN, D), lambda i: (i, 0)),
            pl.BlockSpec((2, BN, HALF), lambda i: (0, i, 0)),
            pl.BlockSpec((2, BN, HALF), lambda i: (0, i, 0)),
        ],
        out_shape=[
            jax.ShapeDtypeStruct((N, D), jnp.float32),
            jax.ShapeDtypeStruct((2, N, HALF), jnp.float32),
            jax.ShapeDtypeStruct((2, N, HALF), jnp.float32),
        ],
    )(h, acc, bias, wl, bl, wr, br)


def _out_body(h_ref, acc_ref, bias_ref, wp_ref, bp_ref, y_ref):
    hn = _combine(h_ref[...], acc_ref, bias_ref[...])
    y_ref[...] = jnp.dot(hn, wp_ref[...],
                         preferred_element_type=jnp.float32) + bp_ref[...]


def _out(h, acc, bias, wpred, bpred):
    return pl.pallas_call(
        _out_body,
        grid=(N // BN,),
        in_specs=[
            pl.BlockSpec((BN, D), lambda i: (i, 0)),
            pl.BlockSpec((2, BN, ROWW), lambda i: (0, i, 0)),
            pl.BlockSpec((1, D), lambda i: (0, 0)),
            pl.BlockSpec((D, NUM_CLASS), lambda i: (0, 0)),
            pl.BlockSpec((1, NUM_CLASS), lambda i: (0, 0)),
        ],
        out_specs=[pl.BlockSpec((BN, NUM_CLASS), lambda i: (i, 0))],
        out_shape=[jax.ShapeDtypeStruct((N, NUM_CLASS), jnp.float32)],
    )(h, acc, bias, wpred, bpred)[0]


def _slab_pieces(total, piece=ZROWS):
    out = []
    off = 0
    while off < total:
        ln = min(piece, total - off)
        out.append((off, ln))
        off += ln
    return out


_GDN = lax.GatherDimensionNumbers(
    offset_dims=(), collapsed_slice_dims=(0,), start_index_map=(0,))


def _lane_perm(v, idx):
    return lax.gather(v, idx.reshape(16, 1), _GDN, (1,),
                      mode=lax.GatherScatterMode.PROMISE_IN_BOUNDS)


def _edge_pass(xl2, xr2, idxm, attm, zeros, base):
    mesh = plsc.VectorSubcoreMesh(core_axis_name="c", subcore_axis_name="s")

    @functools.partial(
        pl.kernel,
        out_type=jax.ShapeDtypeStruct((2, SHROWS, ROWW), jnp.float32),
        mesh=mesh,
        compiler_params=pltpu.CompilerParams(use_tc_tiling_on_sc=False,
                                             needs_layout_passes=False),
        scratch_types=[
            pltpu.VMEM((EPC,), jnp.int32),
            pltpu.VMEM((EPC,), jnp.int32),
            pltpu.VMEM((EPC,), jnp.int32),
            pltpu.VMEM((EPC, HALF), jnp.float32),
            pltpu.VMEM((EPC, HALF), jnp.float32),
            pltpu.VMEM((EPC, ROWW), jnp.float32),
            pltpu.VMEM((HALF,), jnp.float32),
            pltpu.VMEM_SHARED((SHROWS, ROWW), jnp.float32),
            pltpu.SemaphoreType.DMA,
        ],
    )
    def kfn(xl_h, xr_h, idxm_h, attm_h, zeros_h, out_h,
            gsi_v, gdi_v, sdi_v, xj_v, xi_v, rows_v, att_v, shared, sem):
        c = lax.axis_index("c")
        s = lax.axis_index("s")
        pltpu.sync_copy(attm_h.at[c], att_v)

        @pl.when(s == 0)
        def _():
            for off, ln in _slab_pieces(SHROWS):
                pltpu.sync_copy(zeros_h.at[pl.ds(0, ln)],
                                shared.at[pl.ds(off, ln)])
        plsc.subcore_barrier()

        att = att_v[...]
        lane = lax.iota(jnp.int32, 16)
        perm1 = lax.bitwise_xor(lane, 1)
        perm2 = lax.bitwise_xor(lane, 2)
        scol = (lane // C) + HALF
        smask = (lane % C) == 0

        def chunk(g, carry):
            row = s * CHUNKS + g
            pltpu.sync_copy(idxm_h.at[c, row, 0], gsi_v)
            pltpu.sync_copy(idxm_h.at[c, row, 1], gdi_v)
            pltpu.sync_copy(idxm_h.at[c, row, 2], sdi_v)
            cp1 = pltpu.async_copy(xl_h.at[gsi_v], xj_v, sem)
            cp2 = pltpu.async_copy(xr_h.at[gdi_v], xi_v, sem)
            cp1.wait()
            cp2.wait()
            for j in range(EPC // 16):
                d = sdi_v[pl.ds(j * 16, 16)]
                t = d - base
                ok = (t >= 0) & (t < NHALF)
                sdi_v[pl.ds(j * 16, 16)] = jnp.where(ok, t, NHALF)

            def edge(k, carry2):
                xj = xj_v[k]
                xi = xi_v[k]
                t = xi + xj
                t = jnp.maximum(t, t * 0.2)
                u = t * att
                u = u + _lane_perm(u, perm1)
                u = u + _lane_perm(u, perm2)
                p = jnp.exp(u)
                rows_v[k, pl.ds(0, HALF)] = xj * p
                plsc.store_scatter(
                    rows_v, [jnp.full((16,), k, jnp.int32), scol], p,
                    mask=smask)
                return carry2

            lax.fori_loop(0, EPC, edge, 0)
            pltpu.sync_copy(rows_v, shared.at[sdi_v], add=True)
            return carry

        lax.fori_loop(0, CHUNKS, chunk, 0)
        plsc.subcore_barrier()

        @pl.when(s == 0)
        def _():
            for off, ln in _slab_pieces(SHROWS):
                pltpu.sync_copy(shared.at[pl.ds(off, ln)],
                                out_h.at[c, pl.ds(off, ln)])

    return kfn(xl2, xr2, idxm, attm, zeros)


def kernel(x, edge_index, Wp, bp, Wl0, bl0, Wr0, br0, att0, bias0,
           Wl1, bl1, Wr1, br1, att1, bias1, Wpred, bpred):
    src = edge_index[0]
    dst = edge_index[1]
    srcr = src.reshape(EROWS, EPC)
    dstr = dst.reshape(EROWS, EPC)
    idxm = jnp.stack([
        jnp.stack([srcr, dstr, dstr], axis=1),
        jnp.stack([srcr + N, dstr + N, dstr], axis=1),
    ])
    zeros = jnp.zeros((ZROWS, ROWW), jnp.float32)

    h, xl, xr = _proj_in(x, Wp, bp.reshape(1, D), Wl0, bl0.reshape(1, D),
                         Wr0, br0.reshape(1, D))
    xl2, xr2 = xl.reshape(2 * N, HALF), xr.reshape(2 * N, HALF)
    a0_lo = _edge_pass(xl2, xr2, idxm, att0.reshape(2, HALF), zeros, 0)
    a0_hi = _edge_pass(xl2, xr2, idxm, att0.reshape(2, HALF), zeros, NHALF)
    acc0 = jnp.concatenate([a0_lo[:, :NHALF], a0_hi[:, :NHALF]], axis=1)
    h1, xl1, xr1 = _mid(h, acc0, bias0.reshape(1, D), Wl1, bl1.reshape(1, D),
                        Wr1, br1.reshape(1, D))
    xl12, xr12 = xl1.reshape(2 * N, HALF), xr1.reshape(2 * N, HALF)
    a1_lo = _edge_pass(xl12, xr12, idxm, att1.reshape(2, HALF), zeros, 0)
    a1_hi = _edge_pass(xl12, xr12, idxm, att1.reshape(2, HALF), zeros, NHALF)
    acc1 = jnp.concatenate([a1_lo[:, :NHALF], a1_hi[:, :NHALF]], axis=1)
    return _out(h1, acc1, bias1.reshape(1, D), Wpred,
                bpred.reshape(1, NUM_CLASS))
